# baseline (speedup 1.0000x reference)
import functools

import jax
import jax.numpy as jnp
from jax import lax
from jax.experimental import pallas as pl
from jax.experimental.pallas import tpu as pltpu
from jax.experimental.pallas import tpu_sc as plsc

NC = 2
NS = 16
NW = NC * NS

CHUNK = 128


def _make_gather(B, D, n_chunks_w):
    mesh = plsc.VectorSubcoreMesh(core_axis_name="c", subcore_axis_name="s")

    @functools.partial(
        pl.kernel,
        mesh=mesh,
        out_type=jax.ShapeDtypeStruct((B, D), jnp.float32),
        compiler_params=pltpu.CompilerParams(use_tc_tiling_on_sc=False),
        scratch_types=[
            pltpu.VMEM((n_chunks_w, CHUNK), jnp.int32),
            pltpu.VMEM((CHUNK, D), jnp.float32),
            pltpu.SemaphoreType.DMA,
        ],
    )
    def k(x_hbm, idx_hbm, out_hbm, idx_v, rows_v, gsem):
        wid = lax.axis_index("s") * NC + lax.axis_index("c")
        row0 = wid * n_chunks_w
        pltpu.sync_copy(idx_hbm.at[pl.ds(row0, n_chunks_w)], idx_v)

        def body(c, carry):
            pltpu.async_copy(x_hbm.at[idx_v.at[c]], rows_v, gsem).wait()
            pltpu.sync_copy(
                rows_v, out_hbm.at[pl.ds((row0 + c) * CHUNK, CHUNK)])
            return carry

        lax.fori_loop(0, n_chunks_w, body, 0)

    return k


def kernel(x, dim, idx):
    B = idx.shape[0]
    D = x.shape[1]
    idx32 = (idx + dim).astype(jnp.int32)
    n_chunks = B // CHUNK
    idx2d = idx32.reshape(n_chunks, CHUNK)
    n_chunks_w = n_chunks // NW
    out = _make_gather(B, D, n_chunks_w)(x, idx2d)
    return out


# speedup vs baseline: 1.0819x; 1.0819x over previous
import functools

import jax
import jax.numpy as jnp
from jax import lax
from jax.experimental import pallas as pl
from jax.experimental.pallas import tpu as pltpu
from jax.experimental.pallas import tpu_sc as plsc

NC = 2
NS = 16
NW = NC * NS

CHUNK = 128
K = 2
GROWS = K * CHUNK
NBUF = 4


def _make_gather(B, D):
    n_chunks = B // (CHUNK * NW)
    n_groups = n_chunks // K
    mesh = plsc.VectorSubcoreMesh(core_axis_name="c", subcore_axis_name="s")

    @functools.partial(
        pl.kernel,
        mesh=mesh,
        out_type=jax.ShapeDtypeStruct((B, D), jnp.float32),
        compiler_params=pltpu.CompilerParams(use_tc_tiling_on_sc=False),
        scratch_types=(
            [pltpu.VMEM((n_chunks, CHUNK), jnp.int32),
             pltpu.VMEM((NBUF, GROWS, D), jnp.float32)]
            + [pltpu.SemaphoreType.DMA] * (2 * NBUF)
        ),
    )
    def k(x_hbm, idx_hbm, out_hbm, idx_v, rows_v, *sems):
        gsems, ssems = sems[:NBUF], sems[NBUF:]
        wid = lax.axis_index("s") * NC + lax.axis_index("c")
        crow0 = wid * n_chunks
        orow0 = crow0 * CHUNK
        pltpu.sync_copy(idx_hbm.at[pl.ds(crow0, n_chunks)], idx_v)

        def fire(g, b):
            for j in range(K):
                pltpu.async_copy(
                    x_hbm.at[idx_v.at[g * K + j]],
                    rows_v.at[b, pl.ds(j * CHUNK, CHUNK)],
                    gsems[b])

        def drain_gather(b):
            pltpu.make_async_copy(
                x_hbm.at[pl.ds(0, GROWS)], rows_v.at[b], gsems[b]).wait()

        def store(g, b):
            pltpu.async_copy(
                rows_v.at[b],
                out_hbm.at[pl.ds(orow0 + g * GROWS, GROWS)],
                ssems[b])

        def drain_store(b):
            pltpu.make_async_copy(
                rows_v.at[b], out_hbm.at[pl.ds(0, GROWS)], ssems[b]).wait()

        for b in range(NBUF):
            fire(b, b)

        def body(i, carry):
            t = i * NBUF
            for b in range(NBUF):
                g = t + b
                drain_gather(b)
                store(g, b)

                @pl.when(g + NBUF < n_groups)
                def _():
                    drain_store(b)
                    fire(g + NBUF, b)
            return carry

        lax.fori_loop(0, n_groups // NBUF, body, 0)
        for b in range(NBUF):
            drain_store(b)

    return k


def kernel(x, dim, idx):
    B = idx.shape[0]
    D = x.shape[1]
    idx32 = (idx + dim).astype(jnp.int32)
    idx2d = idx32.reshape(B // CHUNK, CHUNK)
    return _make_gather(B, D)(x, idx2d)
